# baseline (speedup 1.0000x reference)
import jax
import jax.numpy as jnp
from jax import lax
from jax.experimental import pallas as pl
from jax.experimental.pallas import tpu as pltpu
from jax.experimental.pallas import tpu_sc as plsc

_EMB = 1024
_NC = 2
_NS = 16
_NW = _NC * _NS
_LANES = 16
_CH = 32


def _sc_body(cpw, tpb, inp_hbm, w_hbm, out_hbm,
             inp_v, pre_v, pos_v, rows0, rows1, rows2,
             gsem0, gsem1, gsem2, wsem0, wsem1, wsem2):
    pre = (tpb - 1) * cpw
    nvec = cpw // _LANES
    nch = cpw // _CH

    c = lax.axis_index("c")
    s = lax.axis_index("s")
    wid = c * _NS + s
    base = wid * cpw
    k = wid % tpb
    row_start = base - k * cpw

    pltpu.sync_copy(inp_hbm.at[pl.ds(base, cpw)], inp_v)
    pltpu.sync_copy(inp_hbm.at[pl.ds(row_start, pre)], pre_v)

    def pre_body(j, acc):
        for u in range(4):
            x = pre_v[pl.ds((j * 4 + u) * _LANES, _LANES)]
            acc = acc + lax.shift_right_logical(x | (-x), 31)
        return acc

    acc0 = jnp.zeros((_LANES,), jnp.int32)
    offset = jnp.sum(lax.fori_loop(0, k * (nvec // 4), pre_body, acc0))

    vpc = _CH // _LANES
    bufs = (rows0, rows1, rows2)
    gsems = (gsem0, gsem1, gsem2)
    wsems = (wsem0, wsem1, wsem2)
    cps = [None, None, None]
    wps = [None, None, None]
    carry = offset
    for ci in range(nch):
        cb = carry
        for u in range(vpc):
            i = ci * vpc + u
            x = inp_v[pl.ds(i * _LANES, _LANES)]
            m = lax.shift_right_logical(x | (-x), 31)
            cs = plsc.cumsum(m) + carry
            pos_v[pl.ds(i * _LANES, _LANES)] = cs * m
            carry = cs[15]
        b = ci % 3
        if wps[b] is not None:
            wps[b].wait()
        dense = (carry - cb) == _CH

        @pl.when(dense)
        def _(b=b, cb=cb):
            pltpu.async_copy(w_hbm.at[pl.ds(cb + 1, _CH)], bufs[b], gsems[b])

        @pl.when(jnp.logical_not(dense))
        def _(b=b, ci=ci):
            pltpu.async_copy(
                w_hbm.at[pos_v.at[pl.ds(ci * _CH, _CH)]], bufs[b], gsems[b])

        cps[b] = pltpu.make_async_copy(
            w_hbm.at[pl.ds(0, _CH)], bufs[b], gsems[b])
        if ci > 0:
            pb = (ci - 1) % 3
            cps[pb].wait()
            wps[pb] = pltpu.async_copy(
                bufs[pb], out_hbm.at[pl.ds(base + (ci - 1) * _CH, _CH)],
                wsems[pb])
    lb = (nch - 1) % 3
    cps[lb].wait()
    wps[lb] = pltpu.async_copy(
        bufs[lb], out_hbm.at[pl.ds(base + (nch - 1) * _CH, _CH)], wsems[lb])
    for b in range(3):
        wps[b].wait()


def kernel(input, weights):
    b, seq_len = input.shape
    n = b * seq_len
    cpw = n // _NW
    tpb = seq_len // cpw
    pre = (tpb - 1) * cpw

    flat = input.reshape(n).astype(jnp.int32)
    mesh = plsc.VectorSubcoreMesh(core_axis_name="c", subcore_axis_name="s")

    import functools
    body = functools.partial(_sc_body, cpw, tpb)
    out = pl.kernel(
        body,
        out_type=jax.ShapeDtypeStruct((n, _EMB), jnp.float32),
        mesh=mesh,
        compiler_params=pltpu.CompilerParams(
            needs_layout_passes=False, use_tc_tiling_on_sc=False),
        scratch_types=[
            pltpu.VMEM((cpw,), jnp.int32),
            pltpu.VMEM((pre,), jnp.int32),
            pltpu.VMEM((cpw,), jnp.int32),
            pltpu.VMEM((_CH, _EMB), jnp.float32),
            pltpu.VMEM((_CH, _EMB), jnp.float32),
            pltpu.VMEM((_CH, _EMB), jnp.float32),
            pltpu.SemaphoreType.DMA,
            pltpu.SemaphoreType.DMA,
            pltpu.SemaphoreType.DMA,
            pltpu.SemaphoreType.DMA,
            pltpu.SemaphoreType.DMA,
            pltpu.SemaphoreType.DMA,
        ],
    )(flat, weights)
    return out.reshape(b, seq_len, _EMB)


# speedup vs baseline: 2.3955x; 2.3955x over previous
import jax
import jax.numpy as jnp
from jax import lax
from jax.experimental import pallas as pl
from jax.experimental.pallas import tpu as pltpu
from jax.experimental.pallas import tpu_sc as plsc

_EMB = 1024
_NC = 2
_NS = 16
_NW = _NC * _NS
_LANES = 16
_CH = 32


def _sc_body(cpw, tpb, inp_hbm, w_hbm, out_hbm,
             inp_v, pre_v, pos_v, rows0, rows1, rows2,
             gsem0, gsem1, gsem2, wsem0, wsem1, wsem2):
    pre = (tpb - 1) * cpw
    nvec = cpw // _LANES
    nch = cpw // _CH

    c = lax.axis_index("c")
    s = lax.axis_index("s")
    wid = c * _NS + s
    base = wid * cpw
    k = wid % tpb
    row_start = base - k * cpw

    icp = pltpu.async_copy(inp_hbm.at[pl.ds(base, cpw)], inp_v, gsem0)
    pcp = pltpu.async_copy(inp_hbm.at[pl.ds(row_start, pre)], pre_v, wsem0)
    pcp.wait()
    icp.wait()

    def pre_body(j, acc):
        for u in range(4):
            x = pre_v[pl.ds((j * 4 + u) * _LANES, _LANES)]
            acc = acc + lax.shift_right_logical(x | (-x), 31)
        return acc

    acc0 = jnp.zeros((_LANES,), jnp.int32)
    offset = jnp.sum(lax.fori_loop(0, k * (nvec // 4), pre_body, acc0))

    vpc = _CH // _LANES
    bufs = (rows0, rows1, rows2)
    gsems = (gsem0, gsem1, gsem2)
    wsems = (wsem0, wsem1, wsem2)
    cps = [None, None, None]
    wps = [None, None, None]
    carry = offset
    for ci in range(nch):
        cb = carry
        for u in range(vpc):
            i = ci * vpc + u
            x = inp_v[pl.ds(i * _LANES, _LANES)]
            m = lax.shift_right_logical(x | (-x), 31)
            cs = plsc.cumsum(m) + carry
            pos_v[pl.ds(i * _LANES, _LANES)] = cs * m
            carry = cs[15]
        b = ci % 3
        if wps[b] is not None:
            wps[b].wait()
        cps[b] = pltpu.async_copy(
            w_hbm.at[pos_v.at[pl.ds(ci * _CH, _CH)]], bufs[b], gsems[b])
        if ci > 0:
            pb = (ci - 1) % 3
            cps[pb].wait()
            wps[pb] = pltpu.async_copy(
                bufs[pb], out_hbm.at[pl.ds(base + (ci - 1) * _CH, _CH)],
                wsems[pb])
    lb = (nch - 1) % 3
    cps[lb].wait()
    wps[lb] = pltpu.async_copy(
        bufs[lb], out_hbm.at[pl.ds(base + (nch - 1) * _CH, _CH)], wsems[lb])
    for b in range(3):
        wps[b].wait()


def kernel(input, weights):
    b, seq_len = input.shape
    n = b * seq_len
    cpw = n // _NW
    tpb = seq_len // cpw
    pre = (tpb - 1) * cpw

    flat = input.reshape(n).astype(jnp.int32)
    mesh = plsc.VectorSubcoreMesh(core_axis_name="c", subcore_axis_name="s")

    import functools
    body = functools.partial(_sc_body, cpw, tpb)
    out = pl.kernel(
        body,
        out_type=jax.ShapeDtypeStruct((n, _EMB), jnp.float32),
        mesh=mesh,
        compiler_params=pltpu.CompilerParams(needs_layout_passes=False),
        scratch_types=[
            pltpu.VMEM((cpw,), jnp.int32),
            pltpu.VMEM((pre,), jnp.int32),
            pltpu.VMEM((cpw,), jnp.int32),
            pltpu.VMEM((_CH, _EMB), jnp.float32),
            pltpu.VMEM((_CH, _EMB), jnp.float32),
            pltpu.VMEM((_CH, _EMB), jnp.float32),
            pltpu.SemaphoreType.DMA,
            pltpu.SemaphoreType.DMA,
            pltpu.SemaphoreType.DMA,
            pltpu.SemaphoreType.DMA,
            pltpu.SemaphoreType.DMA,
            pltpu.SemaphoreType.DMA,
        ],
    )(flat, weights)
    return out.reshape(b, seq_len, _EMB)
